# Initial kernel scaffold; baseline (speedup 1.0000x reference)
import jax
import jax.numpy as jnp
from jax.experimental import pallas as pl


def kernel(input, Wr, br, We, be):
    raise NotImplementedError("write your pallas kernel here")


# speedup vs baseline: 1.3217x; 1.3217x over previous
import functools

import jax
import jax.numpy as jnp
from jax import lax
from jax.experimental import pallas as pl
from jax.experimental.pallas import tpu as pltpu
from jax.experimental.pallas import tpu_sc as plsc

CAP = 300
CPAD = 384
BLK = 128
CHUNK = 256


def _route_kernel(E, G, x_ref, wr_ref, br_ref, slot_ref, g_ref, carry_ref):
    j = pl.program_id(0)

    @pl.when(j == 0)
    def _():
        carry_ref[...] = jnp.zeros_like(carry_ref)

    x = x_ref[...]
    r = jnp.dot(x, wr_ref[...], preferred_element_type=jnp.float32)
    r = r + br_ref[...]
    m = jnp.max(r, axis=1, keepdims=True)
    ex = jnp.exp(r - m)
    p = ex / jnp.sum(ex, axis=1, keepdims=True)
    lane = lax.broadcasted_iota(jnp.int32, p.shape, 1)
    pm = jnp.max(p, axis=1, keepdims=True)
    ids = jnp.min(jnp.where(p == pm, lane, E), axis=1, keepdims=True)
    oh = (lane == ids).astype(jnp.float32)
    ri = lax.broadcasted_iota(jnp.int32, (CHUNK, CHUNK), 0)
    ci = lax.broadcasted_iota(jnp.int32, (CHUNK, CHUNK), 1)
    tri = (ri >= ci).astype(jnp.float32)
    csum = jnp.dot(tri, oh, preferred_element_type=jnp.float32)
    total = carry_ref[...] + csum
    carry_ref[...] = total[CHUNK - 1 : CHUNK, :]
    posf = jnp.sum(total * oh, axis=1, keepdims=True) - 1.0
    pos = posf.astype(jnp.int32)
    keep = pos < CAP
    nvec = j * CHUNK + lax.broadcasted_iota(jnp.int32, (CHUNK, 1), 0)
    slot = ids * CPAD + pos
    slot_ref[...] = jnp.where(keep, slot, G)
    g_ref[...] = jnp.where(keep, slot, G + nvec)


def _route(x, Wr, br):
    N, D = x.shape
    E = br.shape[0]
    G = E * CPAD
    slot, g = pl.pallas_call(
        functools.partial(_route_kernel, E, G),
        grid=(N // CHUNK,),
        in_specs=[
            pl.BlockSpec((CHUNK, D), lambda j: (j, 0)),
            pl.BlockSpec((D, E), lambda j: (0, 0)),
            pl.BlockSpec((1, E), lambda j: (0, 0)),
        ],
        out_specs=[
            pl.BlockSpec((CHUNK, 1), lambda j: (j, 0)),
            pl.BlockSpec((CHUNK, 1), lambda j: (j, 0)),
        ],
        out_shape=[
            jax.ShapeDtypeStruct((N, 1), jnp.int32),
            jax.ShapeDtypeStruct((N, 1), jnp.int32),
        ],
        scratch_shapes=[pltpu.VMEM((1, E), jnp.float32)],
    )(x, Wr, br.reshape(1, E))
    return slot.reshape(N), g.reshape(N)


def _dispatch(x, slot, G, XG_ROWS):
    N, D = x.shape
    info = plsc.get_sparse_core_info()
    NC, NS = info.num_cores, info.num_subcores
    NW = NC * NS
    tpw = N // NW
    mesh = plsc.VectorSubcoreMesh(core_axis_name="c", subcore_axis_name="s")

    @functools.partial(
        pl.kernel,
        mesh=mesh,
        out_type=jax.ShapeDtypeStruct((XG_ROWS, D), jnp.float32),
        scratch_types=[
            pltpu.VMEM((tpw,), jnp.int32),
            pltpu.VMEM((tpw, D), jnp.float32),
            pltpu.SemaphoreType.DMA,
        ],
    )
    def k2(x_hbm, slot_hbm, xg_hbm, slot_v, rows_v, sem):
        wid = lax.axis_index("s") * NC + lax.axis_index("c")
        base = wid * tpw
        pltpu.sync_copy(slot_hbm.at[pl.ds(base, tpw)], slot_v)
        for i in range(tpw // 16):
            sv = slot_v[pl.ds(i * 16, 16)]
            slot_v[pl.ds(i * 16, 16)] = jnp.where(sv >= G, G + wid, sv)
        pltpu.sync_copy(x_hbm.at[pl.ds(base, tpw)], rows_v)
        pltpu.async_copy(rows_v, xg_hbm.at[slot_v], sem).wait()

    return k2(x, slot)


def _ffn_kernel(nf, xg_ref, we_ref, be_ref, xb_ref, y_ref):
    i = pl.program_id(0)

    @pl.when(i < nf)
    def _():
        y = jnp.dot(xg_ref[...], we_ref[0], preferred_element_type=jnp.float32)
        y_ref[...] = y + be_ref[0]

    @pl.when(i >= nf)
    def _():
        y_ref[...] = xb_ref[...]


def _ffn(xg, We, be, x):
    N, D = x.shape
    E = be.shape[0]
    G = E * CPAD
    sub = CPAD // BLK
    nf = E * sub
    nb = N // BLK
    ybig = pl.pallas_call(
        functools.partial(_ffn_kernel, nf),
        grid=(nf + nb,),
        in_specs=[
            pl.BlockSpec((BLK, D), lambda i: (jnp.where(i < nf, i, 0), 0)),
            pl.BlockSpec((1, D, D), lambda i: (jnp.where(i < nf, i // sub, 0), 0, 0)),
            pl.BlockSpec((1, 1, D), lambda i: (jnp.where(i < nf, i // sub, 0), 0, 0)),
            pl.BlockSpec((BLK, D), lambda i: (jnp.where(i < nf, 0, i - nf), 0)),
        ],
        out_specs=pl.BlockSpec((BLK, D), lambda i: (i, 0)),
        out_shape=jax.ShapeDtypeStruct((G + N, D), jnp.float32),
    )(xg, We, be.reshape(E, 1, D), x)
    return ybig


def _combine(ybig, g, N, D):
    info = plsc.get_sparse_core_info()
    NC, NS = info.num_cores, info.num_subcores
    NW = NC * NS
    tpw = N // NW
    mesh = plsc.VectorSubcoreMesh(core_axis_name="c", subcore_axis_name="s")

    @functools.partial(
        pl.kernel,
        mesh=mesh,
        out_type=jax.ShapeDtypeStruct((N, D), jnp.float32),
        scratch_types=[
            pltpu.VMEM((tpw,), jnp.int32),
            pltpu.VMEM((tpw, D), jnp.float32),
            pltpu.SemaphoreType.DMA,
        ],
    )
    def k4(ybig_hbm, g_hbm, out_hbm, g_v, rows_v, sem):
        wid = lax.axis_index("s") * NC + lax.axis_index("c")
        base = wid * tpw
        pltpu.sync_copy(g_hbm.at[pl.ds(base, tpw)], g_v)
        pltpu.async_copy(ybig_hbm.at[g_v], rows_v, sem).wait()
        pltpu.sync_copy(rows_v, out_hbm.at[pl.ds(base, tpw)])

    return k4(ybig, g)


def kernel(input, Wr, br, We, be):
    B, S, D = input.shape
    E = br.shape[0]
    N = B * S
    G = E * CPAD
    XG_ROWS = G + BLK

    x = input.reshape(N, D)
    slot, g = _route(x, Wr, br)
    xg = _dispatch(x, slot, G, XG_ROWS)
    ybig = _ffn(xg, We, be, x)
    out = _combine(ybig, g, N, D)
    return out.reshape(B, S, D)


# speedup vs baseline: 1.6557x; 1.2528x over previous
import functools

import jax
import jax.numpy as jnp
from jax import lax
from jax.experimental import pallas as pl
from jax.experimental.pallas import tpu as pltpu
from jax.experimental.pallas import tpu_sc as plsc

CAP = 300
CPAD = 384
CHUNK = 256


def _route_kernel(E, G, x_ref, wr_ref, br_ref, dst_ref, carry_ref):
    j = pl.program_id(0)

    @pl.when(j == 0)
    def _():
        carry_ref[...] = jnp.zeros_like(carry_ref)

    x = x_ref[...]
    r = jnp.dot(x, wr_ref[...], preferred_element_type=jnp.float32)
    r = r + br_ref[...]
    m = jnp.max(r, axis=1, keepdims=True)
    ex = jnp.exp(r - m)
    p = ex / jnp.sum(ex, axis=1, keepdims=True)
    lane = lax.broadcasted_iota(jnp.int32, p.shape, 1)
    pm = jnp.max(p, axis=1, keepdims=True)
    ids = jnp.min(jnp.where(p == pm, lane, E), axis=1, keepdims=True)
    oh = (lane == ids).astype(jnp.float32)
    ri = lax.broadcasted_iota(jnp.int32, (CHUNK, CHUNK), 0)
    ci = lax.broadcasted_iota(jnp.int32, (CHUNK, CHUNK), 1)
    tri = (ri >= ci).astype(jnp.float32)
    csum = jnp.dot(tri, oh, preferred_element_type=jnp.float32)
    total = carry_ref[...] + csum
    carry_ref[...] = total[CHUNK - 1 : CHUNK, :]
    posf = jnp.sum(total * oh, axis=1, keepdims=True) - 1.0
    pos = posf.astype(jnp.int32)
    keep = pos < CAP
    nvec = j * CHUNK + lax.broadcasted_iota(jnp.int32, (CHUNK, 1), 0)
    dst_ref[...] = jnp.where(keep, ids * CPAD + pos, G + nvec)


def _route(x, Wr, br):
    N, D = x.shape
    E = br.shape[0]
    G = E * CPAD
    dst = pl.pallas_call(
        functools.partial(_route_kernel, E, G),
        grid=(N // CHUNK,),
        in_specs=[
            pl.BlockSpec((CHUNK, D), lambda j: (j, 0)),
            pl.BlockSpec((D, E), lambda j: (0, 0)),
            pl.BlockSpec((1, E), lambda j: (0, 0)),
        ],
        out_specs=pl.BlockSpec((CHUNK, 1), lambda j: (j, 0)),
        out_shape=jax.ShapeDtypeStruct((N, 1), jnp.int32),
        scratch_shapes=[pltpu.VMEM((1, E), jnp.float32)],
    )(x, Wr, br.reshape(1, E))
    return dst.reshape(N)


def _dispatch(x, dst, W_ROWS):
    N, D = x.shape
    info = plsc.get_sparse_core_info()
    NC, NS = info.num_cores, info.num_subcores
    NW = NC * NS
    tpw = N // NW
    mesh = plsc.VectorSubcoreMesh(core_axis_name="c", subcore_axis_name="s")

    @functools.partial(
        pl.kernel,
        mesh=mesh,
        out_type=jax.ShapeDtypeStruct((W_ROWS, D), jnp.float32),
        scratch_types=[
            pltpu.VMEM((tpw,), jnp.int32),
            pltpu.VMEM((tpw, D), jnp.float32),
            pltpu.SemaphoreType.DMA,
        ],
    )
    def k2(x_hbm, dst_hbm, w_hbm, dst_v, rows_v, sem):
        wid = lax.axis_index("s") * NC + lax.axis_index("c")
        base = wid * tpw
        pltpu.sync_copy(dst_hbm.at[pl.ds(base, tpw)], dst_v)
        pltpu.sync_copy(x_hbm.at[pl.ds(base, tpw)], rows_v)
        pltpu.async_copy(rows_v, w_hbm.at[dst_v], sem).wait()

    return k2(x, dst)


def _ffn_kernel(w_ref, we_ref, be_ref, y_ref):
    xb = w_ref[...].astype(jnp.bfloat16)
    y = jnp.dot(xb, we_ref[0], preferred_element_type=jnp.float32)
    y_ref[...] = y + be_ref[0]


def _ffn(w, We16, be):
    R, D = w.shape
    E = be.shape[0]
    ybig = pl.pallas_call(
        _ffn_kernel,
        grid=(E,),
        in_specs=[
            pl.BlockSpec((CPAD, D), lambda i: (i, 0)),
            pl.BlockSpec((1, D, D), lambda i: (i, 0, 0)),
            pl.BlockSpec((1, 1, D), lambda i: (i, 0, 0)),
        ],
        out_specs=pl.BlockSpec((CPAD, D), lambda i: (i, 0)),
        out_shape=jax.ShapeDtypeStruct((R, D), jnp.float32),
        input_output_aliases={0: 0},
    )(w, We16, be.reshape(E, 1, D))
    return ybig


def _combine(ybig, dst, N, D):
    info = plsc.get_sparse_core_info()
    NC, NS = info.num_cores, info.num_subcores
    NW = NC * NS
    tpw = N // NW
    mesh = plsc.VectorSubcoreMesh(core_axis_name="c", subcore_axis_name="s")

    @functools.partial(
        pl.kernel,
        mesh=mesh,
        out_type=jax.ShapeDtypeStruct((N, D), jnp.float32),
        scratch_types=[
            pltpu.VMEM((tpw,), jnp.int32),
            pltpu.VMEM((tpw, D), jnp.float32),
            pltpu.SemaphoreType.DMA,
        ],
    )
    def k4(ybig_hbm, dst_hbm, out_hbm, dst_v, rows_v, sem):
        wid = lax.axis_index("s") * NC + lax.axis_index("c")
        base = wid * tpw
        pltpu.sync_copy(dst_hbm.at[pl.ds(base, tpw)], dst_v)
        pltpu.async_copy(ybig_hbm.at[dst_v], rows_v, sem).wait()
        pltpu.sync_copy(rows_v, out_hbm.at[pl.ds(base, tpw)])

    return k4(ybig, dst)


def kernel(input, Wr, br, We, be):
    B, S, D = input.shape
    E = br.shape[0]
    N = B * S
    G = E * CPAD

    x = input.reshape(N, D)
    dst = _route(x, Wr, br)
    w = _dispatch(x, dst, G + N)
    ybig = _ffn(w, We.astype(jnp.bfloat16), be)
    out = _combine(ybig, dst, N, D)
    return out.reshape(B, S, D)
